# baseline (speedup 1.0000x reference)
import jax
import jax.numpy as jnp
from jax.experimental import pallas as pl
from jax.experimental.pallas import tpu as pltpu

_BM = 1024
_NC = 512


def _chamfer_body(pred_a_ref, gt_a_ref, out_ref, colmin_ref):
    b = pl.program_id(0)
    i = pl.program_id(1)
    n_i = pl.num_programs(1)

    pred_a = pred_a_ref[0]
    gt_a = gt_a_ref[0]

    d2 = jax.lax.dot_general(
        pred_a, gt_a,
        (((0,), (0,)), ((), ())),
        preferred_element_type=jnp.float32,
    )

    del b

    @pl.when(i == 0)
    def _init_out():
        out_ref[...] = jnp.zeros((1, 1, 1), jnp.float32)

    @pl.when(i == 0)
    def _init_colmin():
        colmin_ref[...] = jnp.full(colmin_ref.shape, jnp.inf, jnp.float32)

    row_min = jnp.min(d2, axis=1, keepdims=True)
    out_ref[...] += jnp.sum(row_min, axis=(0, 1), keepdims=True)[None]

    colmin_ref[...] = jnp.minimum(
        colmin_ref[...], jnp.min(d2, axis=0, keepdims=True))

    @pl.when(i == n_i - 1)
    def _finish_cols():
        out_ref[...] += jnp.sum(colmin_ref[...], axis=(0, 1),
                                keepdims=True)[None]


@jax.jit
def kernel(pc_gt, pc_pred):
    B, N, D = pc_gt.shape
    M = pc_pred.shape[1]

    x2 = jnp.sum(pc_pred * pc_pred, axis=2)
    y2 = jnp.sum(pc_gt * pc_gt, axis=2)
    ones_m = jnp.ones((B, 1, M), jnp.float32)
    ones_n = jnp.ones((B, 1, N), jnp.float32)
    zeros_m = jnp.zeros((B, 3, M), jnp.float32)
    zeros_n = jnp.zeros((B, 3, N), jnp.float32)
    pred_a = jnp.concatenate(
        [pc_pred.transpose(0, 2, 1) * (-2.0), x2[:, None, :], ones_m,
         zeros_m], axis=1)
    gt_a = jnp.concatenate(
        [pc_gt.transpose(0, 2, 1), ones_n, y2[:, None, :], zeros_n],
        axis=1)

    def _trisplit(a):
        a0 = jax.lax.optimization_barrier(a.astype(jnp.bfloat16))
        r1 = a - a0.astype(jnp.float32)
        a1 = jax.lax.optimization_barrier(r1.astype(jnp.bfloat16))
        a2 = (r1 - a1.astype(jnp.float32)).astype(jnp.bfloat16)
        return a0, a1, a2

    p0, p1, p2 = _trisplit(pred_a)
    g0, g1, g2 = _trisplit(gt_a)
    pred_k48 = jnp.concatenate([p0, p0, p1, p0, p1, p2], axis=1)
    gt_k48 = jnp.concatenate([g0, g1, g0, g2, g1, g0], axis=1)

    total = pl.pallas_call(
        _chamfer_body,
        grid=(B, M // _BM),
        in_specs=[
            pl.BlockSpec((1, 48, _BM), lambda b, i: (b, 0, i)),
            pl.BlockSpec((1, 48, N), lambda b, i: (b, 0, 0)),
        ],
        out_specs=pl.BlockSpec((1, 1, 1), lambda b, i: (b, 0, 0)),
        out_shape=jax.ShapeDtypeStruct((B, 1, 1), jnp.float32),
        scratch_shapes=[pltpu.VMEM((1, N), jnp.float32)],
        compiler_params=pltpu.CompilerParams(
            dimension_semantics=("parallel", "arbitrary")),
    )(pred_k48, gt_k48)

    return (jnp.sum(total) / (B * M)).astype(jnp.float32)


# speedup vs baseline: 1.1759x; 1.1759x over previous
import jax
import jax.numpy as jnp
from jax.experimental import pallas as pl
from jax.experimental.pallas import tpu as pltpu

_BM = 1024
_NC = 512


def _chamfer_body(pred_a_ref, gt_a_ref, out_ref, colmin_ref):
    b = pl.program_id(0)
    i = pl.program_id(1)
    n_i = pl.num_programs(1)

    pred_a = pred_a_ref[0]
    gt_a = gt_a_ref[0]

    d2 = jax.lax.dot_general(
        pred_a, gt_a,
        (((0,), (0,)), ((), ())),
        preferred_element_type=jnp.float32,
    )

    del b

    d2b = d2.astype(jnp.bfloat16)

    @pl.when(i == 0)
    def _init_out():
        out_ref[...] = jnp.zeros((1, 1, 1), jnp.float32)

    @pl.when(i == 0)
    def _init_colmin():
        colmin_ref[...] = jnp.full(colmin_ref.shape, jnp.inf, jnp.bfloat16)

    row_min = jnp.min(d2b, axis=1, keepdims=True)
    out_ref[...] += jnp.sum(row_min.astype(jnp.float32), axis=(0, 1),
                            keepdims=True)[None]

    colmin_ref[...] = jnp.minimum(
        colmin_ref[...], jnp.min(d2b, axis=0, keepdims=True))

    @pl.when(i == n_i - 1)
    def _finish_cols():
        out_ref[...] += jnp.sum(colmin_ref[...].astype(jnp.float32),
                                axis=(0, 1), keepdims=True)[None]


@jax.jit
def kernel(pc_gt, pc_pred):
    B, N, D = pc_gt.shape
    M = pc_pred.shape[1]

    x2 = jnp.sum(pc_pred * pc_pred, axis=2)
    y2 = jnp.sum(pc_gt * pc_gt, axis=2)
    ones_m = jnp.ones((B, 1, M), jnp.float32)
    ones_n = jnp.ones((B, 1, N), jnp.float32)
    zeros_m = jnp.zeros((B, 3, M), jnp.float32)
    zeros_n = jnp.zeros((B, 3, N), jnp.float32)
    pred_a = jnp.concatenate(
        [pc_pred.transpose(0, 2, 1) * (-2.0), x2[:, None, :], ones_m,
         zeros_m], axis=1)
    gt_a = jnp.concatenate(
        [pc_gt.transpose(0, 2, 1), ones_n, y2[:, None, :], zeros_n],
        axis=1)

    def _trisplit(a):
        a0 = jax.lax.optimization_barrier(a.astype(jnp.bfloat16))
        r1 = a - a0.astype(jnp.float32)
        a1 = jax.lax.optimization_barrier(r1.astype(jnp.bfloat16))
        a2 = (r1 - a1.astype(jnp.float32)).astype(jnp.bfloat16)
        return a0, a1, a2

    p0, p1, p2 = _trisplit(pred_a)
    g0, g1, g2 = _trisplit(gt_a)
    pred_k48 = jnp.concatenate([p0, p0, p1, p0, p1, p2], axis=1)
    gt_k48 = jnp.concatenate([g0, g1, g0, g2, g1, g0], axis=1)

    total = pl.pallas_call(
        _chamfer_body,
        grid=(B, M // _BM),
        in_specs=[
            pl.BlockSpec((1, 48, _BM), lambda b, i: (b, 0, i)),
            pl.BlockSpec((1, 48, N), lambda b, i: (b, 0, 0)),
        ],
        out_specs=pl.BlockSpec((1, 1, 1), lambda b, i: (b, 0, 0)),
        out_shape=jax.ShapeDtypeStruct((B, 1, 1), jnp.float32),
        scratch_shapes=[pltpu.VMEM((1, N), jnp.bfloat16)],
        compiler_params=pltpu.CompilerParams(
            dimension_semantics=("parallel", "arbitrary")),
    )(pred_k48, gt_k48)

    return (jnp.sum(total) / (B * M)).astype(jnp.float32)


# speedup vs baseline: 1.2337x; 1.0492x over previous
import jax
import jax.numpy as jnp
from jax.experimental import pallas as pl
from jax.experimental.pallas import tpu as pltpu

_BM = 2048
_NC = 512


def _chamfer_body(pred_a_ref, gt_a_ref, out_ref, colmin_ref):
    b = pl.program_id(0)
    i = pl.program_id(1)
    n_i = pl.num_programs(1)

    pred_a = pred_a_ref[0]
    gt_a = gt_a_ref[0]

    d2 = jax.lax.dot_general(
        pred_a, gt_a,
        (((0,), (0,)), ((), ())),
        preferred_element_type=jnp.float32,
    )

    del b

    d2b = d2.astype(jnp.bfloat16)

    @pl.when(i == 0)
    def _init_out():
        out_ref[...] = jnp.zeros((1, 1, 1), jnp.float32)

    @pl.when(i == 0)
    def _init_colmin():
        colmin_ref[...] = jnp.full(colmin_ref.shape, jnp.inf, jnp.bfloat16)

    row_min = jnp.min(d2b, axis=1, keepdims=True)
    out_ref[...] += jnp.sum(row_min.astype(jnp.float32), axis=(0, 1),
                            keepdims=True)[None]

    colmin_ref[...] = jnp.minimum(
        colmin_ref[...], jnp.min(d2b, axis=0, keepdims=True))

    @pl.when(i == n_i - 1)
    def _finish_cols():
        out_ref[...] += jnp.sum(colmin_ref[...].astype(jnp.float32),
                                axis=(0, 1), keepdims=True)[None]


@jax.jit
def kernel(pc_gt, pc_pred):
    B, N, D = pc_gt.shape
    M = pc_pred.shape[1]

    x2 = jnp.sum(pc_pred * pc_pred, axis=2)
    y2 = jnp.sum(pc_gt * pc_gt, axis=2)
    ones_m = jnp.ones((B, 1, M), jnp.float32)
    ones_n = jnp.ones((B, 1, N), jnp.float32)
    zeros_m = jnp.zeros((B, 3, M), jnp.float32)
    zeros_n = jnp.zeros((B, 3, N), jnp.float32)
    pred_a = jnp.concatenate(
        [pc_pred.transpose(0, 2, 1) * (-2.0), x2[:, None, :], ones_m,
         zeros_m], axis=1)
    gt_a = jnp.concatenate(
        [pc_gt.transpose(0, 2, 1), ones_n, y2[:, None, :], zeros_n],
        axis=1)

    def _trisplit(a):
        a0 = jax.lax.optimization_barrier(a.astype(jnp.bfloat16))
        r1 = a - a0.astype(jnp.float32)
        a1 = jax.lax.optimization_barrier(r1.astype(jnp.bfloat16))
        a2 = (r1 - a1.astype(jnp.float32)).astype(jnp.bfloat16)
        return a0, a1, a2

    p0, p1, p2 = _trisplit(pred_a)
    g0, g1, g2 = _trisplit(gt_a)
    pred_k48 = jnp.concatenate([p0, p0, p1, p0, p1, p2], axis=1)
    gt_k48 = jnp.concatenate([g0, g1, g0, g2, g1, g0], axis=1)

    total = pl.pallas_call(
        _chamfer_body,
        grid=(B, M // _BM),
        in_specs=[
            pl.BlockSpec((1, 48, _BM), lambda b, i: (b, 0, i)),
            pl.BlockSpec((1, 48, N), lambda b, i: (b, 0, 0)),
        ],
        out_specs=pl.BlockSpec((1, 1, 1), lambda b, i: (b, 0, 0)),
        out_shape=jax.ShapeDtypeStruct((B, 1, 1), jnp.float32),
        scratch_shapes=[pltpu.VMEM((1, N), jnp.bfloat16)],
        compiler_params=pltpu.CompilerParams(
            dimension_semantics=("parallel", "arbitrary")),
    )(pred_k48, gt_k48)

    return (jnp.sum(total) / (B * M)).astype(jnp.float32)


# speedup vs baseline: 1.5550x; 1.2604x over previous
import jax
import jax.numpy as jnp
from jax.experimental import pallas as pl
from jax.experimental.pallas import tpu as pltpu

_BM = 4096
_NC = 512


def _chamfer_body(pred_a_ref, gt_a_ref, out_ref, colmin_ref):
    b = pl.program_id(0)
    i = pl.program_id(1)
    n_i = pl.num_programs(1)

    pred_a = pred_a_ref[0]
    gt_a = gt_a_ref[0]

    d2 = jax.lax.dot_general(
        pred_a, gt_a,
        (((0,), (0,)), ((), ())),
        preferred_element_type=jnp.float32,
    )

    del b

    d2b = d2.astype(jnp.bfloat16)

    @pl.when(i == 0)
    def _init_out():
        out_ref[...] = jnp.zeros((1, 1, 1), jnp.float32)

    @pl.when(i == 0)
    def _init_colmin():
        colmin_ref[...] = jnp.full(colmin_ref.shape, jnp.inf, jnp.bfloat16)

    row_min = jnp.min(d2b, axis=1, keepdims=True)
    out_ref[...] += jnp.sum(row_min.astype(jnp.float32), axis=(0, 1),
                            keepdims=True)[None]

    colmin_ref[...] = jnp.minimum(
        colmin_ref[...], jnp.min(d2b, axis=0, keepdims=True))

    @pl.when(i == n_i - 1)
    def _finish_cols():
        out_ref[...] += jnp.sum(colmin_ref[...].astype(jnp.float32),
                                axis=(0, 1), keepdims=True)[None]


@jax.jit
def kernel(pc_gt, pc_pred):
    B, N, D = pc_gt.shape
    M = pc_pred.shape[1]

    x2 = jnp.sum(pc_pred * pc_pred, axis=2)
    y2 = jnp.sum(pc_gt * pc_gt, axis=2)
    ones_m = jnp.ones((B, 1, M), jnp.float32)
    ones_n = jnp.ones((B, 1, N), jnp.float32)
    zeros_m = jnp.zeros((B, 3, M), jnp.float32)
    zeros_n = jnp.zeros((B, 3, N), jnp.float32)
    pred_a = jnp.concatenate(
        [pc_pred.transpose(0, 2, 1) * (-2.0), x2[:, None, :], ones_m,
         zeros_m], axis=1)
    gt_a = jnp.concatenate(
        [pc_gt.transpose(0, 2, 1), ones_n, y2[:, None, :], zeros_n],
        axis=1)

    def _trisplit(a):
        a0 = jax.lax.optimization_barrier(a.astype(jnp.bfloat16))
        r1 = a - a0.astype(jnp.float32)
        a1 = jax.lax.optimization_barrier(r1.astype(jnp.bfloat16))
        a2 = (r1 - a1.astype(jnp.float32)).astype(jnp.bfloat16)
        return a0, a1, a2

    p0, p1, p2 = _trisplit(pred_a)
    g0, g1, g2 = _trisplit(gt_a)
    pred_k48 = jnp.concatenate([p0, p0, p1, p0, p1, p2], axis=1)
    gt_k48 = jnp.concatenate([g0, g1, g0, g2, g1, g0], axis=1)

    total = pl.pallas_call(
        _chamfer_body,
        grid=(B, M // _BM),
        in_specs=[
            pl.BlockSpec((1, 48, _BM), lambda b, i: (b, 0, i)),
            pl.BlockSpec((1, 48, N), lambda b, i: (b, 0, 0)),
        ],
        out_specs=pl.BlockSpec((1, 1, 1), lambda b, i: (b, 0, 0)),
        out_shape=jax.ShapeDtypeStruct((B, 1, 1), jnp.float32),
        scratch_shapes=[pltpu.VMEM((1, N), jnp.bfloat16)],
        compiler_params=pltpu.CompilerParams(
            dimension_semantics=("parallel", "arbitrary")),
    )(pred_k48, gt_k48)

    return (jnp.sum(total) / (B * M)).astype(jnp.float32)
